# baseline (speedup 1.0000x reference)
import functools

import jax
import jax.numpy as jnp
from jax import lax
from jax.experimental import pallas as pl
from jax.experimental.pallas import tpu as pltpu
from jax.experimental.pallas import tpu_sc as plsc

N = 10000
E = 320000
D = 128
H = 256
C = 10
G = 128

NC = 2
NS = 16
B = 112
KCH = 184
W = 8
NWIN = KCH // W
EPAD = NS * KCH * B

RCH = 80
NCH_N = N // RCH
NSP = 10080
NCH_SP = NSP // RCH
JMAX = 8

NB = 400
NBLK = N // NB


def _tile_chunks(tid, nch, copy_fn):
    for j in range(JMAX):
        c = tid + NS * j
        if j < (nch // NS):
            copy_fn(pl.multiple_of(c * RCH, 8))
        else:
            @pl.when(c < nch)
            def _():
                copy_fn(pl.multiple_of(c * RCH, 8))


@functools.cache
def _sc_kernels():
    mesh = plsc.VectorSubcoreMesh(core_axis_name="c", subcore_axis_name="s")

    @functools.partial(
        pl.kernel,
        out_type=jax.ShapeDtypeStruct((NC, N, D), jnp.float32),
        mesh=mesh,
        scratch_types=[
            pltpu.VMEM((W, B), jnp.int32),
            pltpu.VMEM((B, D), jnp.float32),
            pltpu.VMEM((B, D), jnp.float32),
            pltpu.VMEM_SHARED((NSP, D), jnp.float32),
            pltpu.SemaphoreType.DMA,
            pltpu.SemaphoreType.DMA,
        ],
    )
    def deg_pass(dst3, ones_hbm, zeros_hbm, out, idxd, ob0, ob1, sp,
                 sem0, sem1):
        cid = lax.axis_index("c")
        tid = lax.axis_index("s")
        bufs = (ob0, ob1)
        sems = (sem0, sem1)
        pltpu.sync_copy(zeros_hbm, ob0.at[pl.ds(0, RCH)])
        _tile_chunks(tid, NCH_SP,
                     lambda r: pltpu.sync_copy(ob0.at[pl.ds(0, RCH)],
                                               sp.at[pl.ds(r, RCH)]))
        plsc.subcore_barrier()

        def win_body(w, carry):
            wb = pl.multiple_of(w * W, 8)
            pltpu.sync_copy(dst3.at[tid, pl.ds(wb, W)], idxd)
            cp = pltpu.async_copy(ones_hbm.at[idxd.at[0]], bufs[0], sems[0])
            for k in range(W):
                nxt = None
                if k + 1 < W:
                    j = (k + 1) % 2
                    nxt = pltpu.async_copy(ones_hbm.at[idxd.at[k + 1]],
                                           bufs[j], sems[j])
                cp.wait()
                pltpu.sync_copy(bufs[k % 2], sp.at[idxd.at[k]], add=True)
                cp = nxt
            return carry

        @pl.when(cid == 0)
        def _():
            lax.fori_loop(0, NWIN // 2, win_body, 0)

        @pl.when(cid == 1)
        def _():
            lax.fori_loop(NWIN // 2, NWIN, win_body, 0)
        plsc.subcore_barrier()

        def wout(r):
            pltpu.sync_copy(sp.at[pl.ds(r, RCH)], ob0.at[pl.ds(0, RCH)])
            pltpu.sync_copy(ob0.at[pl.ds(0, RCH)], out.at[cid, pl.ds(r, RCH)])

        _tile_chunks(tid, NCH_N, wout)

    @functools.partial(
        pl.kernel,
        out_type=(jax.ShapeDtypeStruct((N, D), jnp.float32),
                  jax.ShapeDtypeStruct((N, D), jnp.float32)),
        mesh=mesh,
        scratch_types=[
            pltpu.VMEM((W, B), jnp.int32),
            pltpu.VMEM((W, B), jnp.int32),
            pltpu.VMEM((B, D), jnp.float32),
            pltpu.VMEM((B, D), jnp.float32),
            pltpu.VMEM((B, D), jnp.float32),
            pltpu.VMEM_SHARED((NSP, D), jnp.float32),
            pltpu.SemaphoreType.DMA,
            pltpu.SemaphoreType.DMA,
            pltpu.SemaphoreType.DMA,
            pltpu.SemaphoreType.DMA,
            pltpu.SemaphoreType.DMA,
            pltpu.SemaphoreType.DMA,
        ],
    )
    def edge_pass(src3, dst3, h0, h1, zeros_hbm, out0, out1,
                  idxs, idxd, rb0, rb1, rb2, sp,
                  gs0, gs1, gs2, ss0, ss1, ss2):
        cid = lax.axis_index("c")
        tid = lax.axis_index("s")
        bufs = (rb0, rb1, rb2)
        gsems = (gs0, gs1, gs2)
        ssems = (ss0, ss1, ss2)

        def run(h_ref, out_ref):
            pltpu.sync_copy(zeros_hbm, rb0.at[pl.ds(0, RCH)])
            _tile_chunks(tid, NCH_SP,
                         lambda r: pltpu.sync_copy(rb0.at[pl.ds(0, RCH)],
                                                   sp.at[pl.ds(r, RCH)]))
            plsc.subcore_barrier()

            def win_body(w, carry):
                wb = pl.multiple_of(w * W, 8)
                pltpu.sync_copy(src3.at[tid, pl.ds(wb, W)], idxs)
                pltpu.sync_copy(dst3.at[tid, pl.ds(wb, W)], idxd)
                gat = [None] * W
                sca = [None] * W
                for k in range(2):
                    gat[k] = pltpu.async_copy(h_ref.at[idxs.at[k]],
                                              bufs[k % 3], gsems[k % 3])
                for k in range(W):
                    gat[k].wait()
                    sca[k] = pltpu.async_copy(bufs[k % 3],
                                              sp.at[idxd.at[k]],
                                              ssems[k % 3], add=True)
                    if k + 2 < W:
                        if k - 1 >= 0:
                            sca[k - 1].wait()
                        j = (k + 2) % 3
                        gat[k + 2] = pltpu.async_copy(
                            h_ref.at[idxs.at[k + 2]], bufs[j], gsems[j])
                for k in range(W - 3, W):
                    if k >= 0:
                        sca[k].wait()
                return carry

            lax.fori_loop(0, NWIN, win_body, 0)
            plsc.subcore_barrier()

            def wout(r):
                pltpu.sync_copy(sp.at[pl.ds(r, RCH)], rb0.at[pl.ds(0, RCH)])
                pltpu.sync_copy(rb0.at[pl.ds(0, RCH)],
                                out_ref.at[pl.ds(r, RCH)])

            _tile_chunks(tid, NCH_N, wout)

        @pl.when(cid == 0)
        def _():
            run(h0, out0)

        @pl.when(cid == 1)
        def _():
            run(h1, out1)

    return deg_pass, edge_pass


def _k1_body(x, w1, d0, d1, o0, o1):
    dinv = lax.rsqrt(d0[:, :1] + d1[:, :1] + 1.0)
    hs = jnp.dot(x[...], w1[...], preferred_element_type=jnp.float32) * dinv
    o0[...] = hs[:, :D]
    o1[...] = hs[:, D:]


def _k2_body(s0, s1, h0, h1, d0, d1, w2, b1, o0, o1):
    dinv = lax.rsqrt(d0[:, :1] + d1[:, :1] + 1.0)
    agg = jnp.concatenate([s0[...] + h0[...], s1[...] + h1[...]], axis=1)
    h = jax.nn.relu(dinv * agg + b1[...])
    hs = jnp.dot(h, w2[...], preferred_element_type=jnp.float32) * dinv
    o0[...] = hs[:, :D]
    o1[...] = hs[:, D:]


def _k3_body(s0, s1, h0, h1, d0, d1, b2, bat, wf1, bf1, wf2, bf2,
             out, sums, cnts):
    i = pl.program_id(0)

    @pl.when(i == 0)
    def _():
        sums[...] = jnp.zeros_like(sums)
        cnts[...] = jnp.zeros_like(cnts)

    dinv = lax.rsqrt(d0[:, :1] + d1[:, :1] + 1.0)
    agg = jnp.concatenate([s0[...] + h0[...], s1[...] + h1[...]], axis=1)
    h2 = jax.nn.relu(dinv * agg + b2[...])
    bids = bat[...].reshape(1, NB)
    oh = (lax.broadcasted_iota(jnp.int32, (G, NB), 0) == bids)
    oh = oh.astype(jnp.float32)
    sums[...] += jnp.dot(oh, h2, preferred_element_type=jnp.float32)
    cnts[...] += jnp.sum(oh, axis=1, keepdims=True)

    @pl.when(i == NBLK - 1)
    def _():
        pooled = sums[...] / jnp.maximum(cnts[...], 1.0)
        f = jax.nn.relu(
            jnp.dot(pooled, wf1[...], preferred_element_type=jnp.float32)
            + bf1[...])
        out[...] = (jnp.dot(f, wf2[...], preferred_element_type=jnp.float32)
                    + bf2[...])


def _row_spec(width):
    return pl.BlockSpec((NB, width), lambda i: (i, 0))


def _full_spec(shape):
    nd = len(shape)
    return pl.BlockSpec(shape, lambda i: (0,) * nd)


def kernel(x, edge_index, batch, W1, b1, W2, b2, Wf1, bf1, Wf2, bf2):
    src = edge_index[0]
    dst = edge_index[1]
    pad = EPAD - E
    src3 = jnp.concatenate([src, jnp.zeros((pad,), jnp.int32)]).reshape(
        NS, KCH, B)
    dst3 = jnp.concatenate([dst, jnp.full((pad,), N, jnp.int32)]).reshape(
        NS, KCH, B)
    onesD = jnp.ones((NSP, D), jnp.float32)
    zerosD = jnp.zeros((RCH, D), jnp.float32)

    _deg_pass, _edge_pass = _sc_kernels()
    deg = _deg_pass(dst3, onesD, zerosD)
    deg0, deg1 = deg[0], deg[1]

    k1 = pl.pallas_call(
        _k1_body,
        grid=(NBLK,),
        in_specs=[_row_spec(D), _full_spec((D, H)), _row_spec(D),
                  _row_spec(D)],
        out_specs=(_row_spec(D), _row_spec(D)),
        out_shape=(jax.ShapeDtypeStruct((N, D), jnp.float32),
                   jax.ShapeDtypeStruct((N, D), jnp.float32)),
    )
    hs0, hs1 = k1(x, W1, deg0, deg1)

    a0, a1 = _edge_pass(src3, dst3, hs0, hs1, zerosD)

    k2 = pl.pallas_call(
        _k2_body,
        grid=(NBLK,),
        in_specs=[_row_spec(D), _row_spec(D), _row_spec(D), _row_spec(D),
                  _row_spec(D), _row_spec(D),
                  _full_spec((H, H)), _full_spec((1, H))],
        out_specs=(_row_spec(D), _row_spec(D)),
        out_shape=(jax.ShapeDtypeStruct((N, D), jnp.float32),
                   jax.ShapeDtypeStruct((N, D), jnp.float32)),
    )
    t0, t1 = k2(a0, a1, hs0, hs1, deg0, deg1, W2, b1.reshape(1, H))

    u0, u1 = _edge_pass(src3, dst3, t0, t1, zerosD)

    batch3 = batch.reshape(NBLK, 1, NB)
    k3 = pl.pallas_call(
        _k3_body,
        grid=(NBLK,),
        in_specs=[_row_spec(D), _row_spec(D), _row_spec(D), _row_spec(D),
                  _row_spec(D), _row_spec(D),
                  _full_spec((1, H)),
                  pl.BlockSpec((1, 1, NB), lambda i: (i, 0, 0)),
                  _full_spec((H, H)), _full_spec((1, H)),
                  _full_spec((H, C)), _full_spec((1, C))],
        out_specs=_full_spec((G, C)),
        out_shape=jax.ShapeDtypeStruct((G, C), jnp.float32),
        scratch_shapes=[pltpu.VMEM((G, H), jnp.float32),
                        pltpu.VMEM((G, 1), jnp.float32)],
    )
    out = k3(u0, u1, t0, t1, deg0, deg1, b2.reshape(1, H), batch3,
             Wf1, bf1.reshape(1, H), Wf2, bf2.reshape(1, C))
    return out


# speedup vs baseline: 1.3494x; 1.3494x over previous
import functools

import jax
import jax.numpy as jnp
from jax import lax
from jax.experimental import pallas as pl
from jax.experimental.pallas import tpu as pltpu
from jax.experimental.pallas import tpu_sc as plsc

N = 10000
E = 320000
D = 128
H = 256
C = 10
G = 128

NC = 2
NS = 16
B = 128
KCH = 160
W = 16
NWIN = KCH // W
EPAD = NS * KCH * B

RCH = 80
NCH_N = N // RCH
NSP = 10080
NCH_SP = NSP // RCH
JMAX = 8

NB = 400
NBLK = N // NB


def _tile_chunks(tid, nch, copy_fn):
    for j in range(JMAX):
        c = tid + NS * j
        if j < (nch // NS):
            copy_fn(pl.multiple_of(c * RCH, 8))
        else:
            @pl.when(c < nch)
            def _():
                copy_fn(pl.multiple_of(c * RCH, 8))


@functools.cache
def _sc_kernels():
    mesh = plsc.VectorSubcoreMesh(core_axis_name="c", subcore_axis_name="s")

    @functools.partial(
        pl.kernel,
        out_type=jax.ShapeDtypeStruct((NC, N, D), jnp.float32),
        mesh=mesh,
        scratch_types=[
            pltpu.VMEM((W, B), jnp.int32),
            pltpu.VMEM((B, D), jnp.float32),
            pltpu.VMEM((B, D), jnp.float32),
            pltpu.VMEM_SHARED((NSP, D), jnp.float32),
            pltpu.SemaphoreType.DMA,
            pltpu.SemaphoreType.DMA,
        ],
    )
    def deg_pass(dst3, ones_hbm, zeros_hbm, out, idxd, ob0, ob1, sp,
                 sem0, sem1):
        cid = lax.axis_index("c")
        tid = lax.axis_index("s")
        bufs = (ob0, ob1)
        sems = (sem0, sem1)
        pltpu.sync_copy(zeros_hbm, ob0.at[pl.ds(0, RCH)])
        _tile_chunks(tid, NCH_SP,
                     lambda r: pltpu.sync_copy(ob0.at[pl.ds(0, RCH)],
                                               sp.at[pl.ds(r, RCH)]))
        plsc.subcore_barrier()

        def win_body(w, carry):
            wb = pl.multiple_of(w * W, 8)
            pltpu.sync_copy(dst3.at[tid, pl.ds(wb, W)], idxd)
            cp = pltpu.async_copy(ones_hbm.at[idxd.at[0]], bufs[0], sems[0])
            for k in range(W):
                nxt = None
                if k + 1 < W:
                    j = (k + 1) % 2
                    nxt = pltpu.async_copy(ones_hbm.at[idxd.at[k + 1]],
                                           bufs[j], sems[j])
                cp.wait()
                pltpu.sync_copy(bufs[k % 2], sp.at[idxd.at[k]], add=True)
                cp = nxt
            return carry

        @pl.when(cid == 0)
        def _():
            lax.fori_loop(0, NWIN // 2, win_body, 0)

        @pl.when(cid == 1)
        def _():
            lax.fori_loop(NWIN // 2, NWIN, win_body, 0)
        plsc.subcore_barrier()

        def wout(r):
            pltpu.sync_copy(sp.at[pl.ds(r, RCH)], ob0.at[pl.ds(0, RCH)])
            pltpu.sync_copy(ob0.at[pl.ds(0, RCH)], out.at[cid, pl.ds(r, RCH)])

        _tile_chunks(tid, NCH_N, wout)

    @functools.partial(
        pl.kernel,
        out_type=(jax.ShapeDtypeStruct((N, D), jnp.float32),
                  jax.ShapeDtypeStruct((N, D), jnp.float32)),
        mesh=mesh,
        scratch_types=[
            pltpu.VMEM((W, B), jnp.int32),
            pltpu.VMEM((W, B), jnp.int32),
            pltpu.VMEM((B, D), jnp.float32),
            pltpu.VMEM((B, D), jnp.float32),
            pltpu.VMEM_SHARED((NSP, D), jnp.float32),
            pltpu.SemaphoreType.DMA,
            pltpu.SemaphoreType.DMA,
        ],
    )
    def edge_pass(src3, dst3, h0, h1, zeros_hbm, out0, out1,
                  idxs, idxd, rb0, rb1, sp, sem0, sem1):
        cid = lax.axis_index("c")
        tid = lax.axis_index("s")
        bufs = (rb0, rb1)
        sems = (sem0, sem1)

        def run(h_ref, out_ref):
            pltpu.sync_copy(zeros_hbm, rb0.at[pl.ds(0, RCH)])
            _tile_chunks(tid, NCH_SP,
                         lambda r: pltpu.sync_copy(rb0.at[pl.ds(0, RCH)],
                                                   sp.at[pl.ds(r, RCH)]))
            plsc.subcore_barrier()

            def win_body(w, carry):
                wb = pl.multiple_of(w * W, 8)
                pltpu.sync_copy(src3.at[tid, pl.ds(wb, W)], idxs)
                pltpu.sync_copy(dst3.at[tid, pl.ds(wb, W)], idxd)
                cp = pltpu.async_copy(h_ref.at[idxs.at[0]], bufs[0], sems[0])
                for k in range(W):
                    nxt = None
                    if k + 1 < W:
                        j = (k + 1) % 2
                        nxt = pltpu.async_copy(h_ref.at[idxs.at[k + 1]],
                                               bufs[j], sems[j])
                    cp.wait()
                    pltpu.sync_copy(bufs[k % 2], sp.at[idxd.at[k]], add=True)
                    cp = nxt
                return carry

            lax.fori_loop(0, NWIN, win_body, 0)
            plsc.subcore_barrier()

            def wout(r):
                pltpu.sync_copy(sp.at[pl.ds(r, RCH)], rb0.at[pl.ds(0, RCH)])
                pltpu.sync_copy(rb0.at[pl.ds(0, RCH)],
                                out_ref.at[pl.ds(r, RCH)])

            _tile_chunks(tid, NCH_N, wout)

        @pl.when(cid == 0)
        def _():
            run(h0, out0)

        @pl.when(cid == 1)
        def _():
            run(h1, out1)

    return deg_pass, edge_pass


def _k1_body(x, w1, d0, d1, o0, o1):
    dinv = lax.rsqrt(d0[:, :1] + d1[:, :1] + 1.0)
    hs = jnp.dot(x[...], w1[...], preferred_element_type=jnp.float32) * dinv
    o0[...] = hs[:, :D]
    o1[...] = hs[:, D:]


def _k2_body(s0, s1, h0, h1, d0, d1, w2, b1, o0, o1):
    dinv = lax.rsqrt(d0[:, :1] + d1[:, :1] + 1.0)
    agg = jnp.concatenate([s0[...] + h0[...], s1[...] + h1[...]], axis=1)
    h = jax.nn.relu(dinv * agg + b1[...])
    hs = jnp.dot(h, w2[...], preferred_element_type=jnp.float32) * dinv
    o0[...] = hs[:, :D]
    o1[...] = hs[:, D:]


def _k3_body(s0, s1, h0, h1, d0, d1, b2, bat, wf1, bf1, wf2, bf2,
             out, sums, cnts):
    i = pl.program_id(0)

    @pl.when(i == 0)
    def _():
        sums[...] = jnp.zeros_like(sums)
        cnts[...] = jnp.zeros_like(cnts)

    dinv = lax.rsqrt(d0[:, :1] + d1[:, :1] + 1.0)
    agg = jnp.concatenate([s0[...] + h0[...], s1[...] + h1[...]], axis=1)
    h2 = jax.nn.relu(dinv * agg + b2[...])
    bids = bat[...].reshape(1, NB)
    oh = (lax.broadcasted_iota(jnp.int32, (G, NB), 0) == bids)
    oh = oh.astype(jnp.float32)
    sums[...] += jnp.dot(oh, h2, preferred_element_type=jnp.float32)
    cnts[...] += jnp.sum(oh, axis=1, keepdims=True)

    @pl.when(i == NBLK - 1)
    def _():
        pooled = sums[...] / jnp.maximum(cnts[...], 1.0)
        f = jax.nn.relu(
            jnp.dot(pooled, wf1[...], preferred_element_type=jnp.float32)
            + bf1[...])
        out[...] = (jnp.dot(f, wf2[...], preferred_element_type=jnp.float32)
                    + bf2[...])


def _row_spec(width):
    return pl.BlockSpec((NB, width), lambda i: (i, 0))


def _full_spec(shape):
    nd = len(shape)
    return pl.BlockSpec(shape, lambda i: (0,) * nd)


def kernel(x, edge_index, batch, W1, b1, W2, b2, Wf1, bf1, Wf2, bf2):
    src = edge_index[0]
    dst = edge_index[1]
    pad = EPAD - E
    src3 = jnp.concatenate([src, jnp.zeros((pad,), jnp.int32)]).reshape(
        NS, KCH, B)
    dst3 = jnp.concatenate([dst, jnp.full((pad,), N, jnp.int32)]).reshape(
        NS, KCH, B)
    onesD = jnp.ones((NSP, D), jnp.float32)
    zerosD = jnp.zeros((RCH, D), jnp.float32)

    _deg_pass, _edge_pass = _sc_kernels()
    deg = _deg_pass(dst3, onesD, zerosD)
    deg0, deg1 = deg[0], deg[1]

    k1 = pl.pallas_call(
        _k1_body,
        grid=(NBLK,),
        in_specs=[_row_spec(D), _full_spec((D, H)), _row_spec(D),
                  _row_spec(D)],
        out_specs=(_row_spec(D), _row_spec(D)),
        out_shape=(jax.ShapeDtypeStruct((N, D), jnp.float32),
                   jax.ShapeDtypeStruct((N, D), jnp.float32)),
    )
    hs0, hs1 = k1(x, W1, deg0, deg1)

    a0, a1 = _edge_pass(src3, dst3, hs0, hs1, zerosD)

    k2 = pl.pallas_call(
        _k2_body,
        grid=(NBLK,),
        in_specs=[_row_spec(D), _row_spec(D), _row_spec(D), _row_spec(D),
                  _row_spec(D), _row_spec(D),
                  _full_spec((H, H)), _full_spec((1, H))],
        out_specs=(_row_spec(D), _row_spec(D)),
        out_shape=(jax.ShapeDtypeStruct((N, D), jnp.float32),
                   jax.ShapeDtypeStruct((N, D), jnp.float32)),
    )
    t0, t1 = k2(a0, a1, hs0, hs1, deg0, deg1, W2, b1.reshape(1, H))

    u0, u1 = _edge_pass(src3, dst3, t0, t1, zerosD)

    batch3 = batch.reshape(NBLK, 1, NB)
    k3 = pl.pallas_call(
        _k3_body,
        grid=(NBLK,),
        in_specs=[_row_spec(D), _row_spec(D), _row_spec(D), _row_spec(D),
                  _row_spec(D), _row_spec(D),
                  _full_spec((1, H)),
                  pl.BlockSpec((1, 1, NB), lambda i: (i, 0, 0)),
                  _full_spec((H, H)), _full_spec((1, H)),
                  _full_spec((H, C)), _full_spec((1, C))],
        out_specs=_full_spec((G, C)),
        out_shape=jax.ShapeDtypeStruct((G, C), jnp.float32),
        scratch_shapes=[pltpu.VMEM((G, H), jnp.float32),
                        pltpu.VMEM((G, 1), jnp.float32)],
    )
    out = k3(u0, u1, t0, t1, deg0, deg1, b2.reshape(1, H), batch3,
             Wf1, bf1.reshape(1, H), Wf2, bf2.reshape(1, C))
    return out
